# Initial kernel scaffold; baseline (speedup 1.0000x reference)
import jax
import jax.numpy as jnp
from jax.experimental import pallas as pl


def kernel(current_values, global_values, current_coords, global_coords, relative_origin):
    raise NotImplementedError("write your pallas kernel here")


# speedup vs baseline: 4.7255x; 4.7255x over previous
import functools

import jax
import jax.numpy as jnp
from jax import lax
from jax.experimental import pallas as pl
from jax.experimental.pallas import tpu as pltpu
from jax.experimental.pallas import tpu_sc as plsc

V = 96 * 96 * 96
NG = 524288
NC = 262144
CH = 16

NW = 32
GB = NG // NW
CB = NC // NW
SLAB = V // NW
WTBL = 32768

GSTAGE = GB + 16 * NW
CSTAGE = CB + 16 * NW
SEGCHUNK = 2048
GROW = GSTAGE + SEGCHUNK
CROW = CSTAGE + SEGCHUNK
OCH = SLAB // 16
NDUMMY = 2048
SENT = 32767

_mesh = lambda: plsc.VectorSubcoreMesh(core_axis_name="c", subcore_axis_name="s")
_cparams = lambda: pltpu.CompilerParams(needs_layout_passes=False,
                                        use_tc_tiling_on_sc=False)


def _iota():
    return lax.iota(jnp.int32, 16)


def _splat(x):
    return jnp.broadcast_to(jnp.asarray(x, jnp.int32), (16,))


@functools.partial(
    pl.kernel,
    mesh=_mesh(),
    compiler_params=_cparams(),
    out_type=(
        jax.ShapeDtypeStruct((NW * GROW,), jnp.int32),
        jax.ShapeDtypeStruct((NW * CROW,), jnp.int32),
        jax.ShapeDtypeStruct((2 * NW * 32,), jnp.int32),
    ),
    scratch_types=[
        pltpu.VMEM((GB * 3,), jnp.int32),
        pltpu.VMEM((GB,), jnp.int32),
        pltpu.VMEM((GSTAGE,), jnp.int32),
        pltpu.VMEM((512,), jnp.int32),
        pltpu.VMEM((512,), jnp.int32),
        pltpu.VMEM((32,), jnp.int32),
        pltpu.VMEM((48,), jnp.int32),
    ],
)
def _kernel_a(cur_coords, glob_coords, origin_b, part_g, part_c, counts,
              coords_v, rbuf_v, stage_v, hist_v, colptr_v, counts_v, origin_v):
    w = lax.axis_index("s") * 2 + lax.axis_index("c")
    lane = _iota()
    zeros16 = _splat(0)
    pltpu.sync_copy(origin_b, origin_v)

    def run_kind(kind, B, BSTAGE, ROW, coords_hbm, part_hbm, shift_origin):
        nvr = B // 16
        pltpu.sync_copy(coords_hbm.at[pl.ds(pl.multiple_of(w * B * 3, 8), B * 3)],
                        coords_v.at[pl.ds(0, B * 3)])

        if shift_origin:
            ox = origin_v[pl.ds(0, 16)]
            oy = origin_v[pl.ds(16, 16)]
            oz = origin_v[pl.ds(32, 16)]

        def zero_hist(i, _):
            hist_v[pl.ds(i * 16, 16)] = zeros16
            return 0
        lax.fori_loop(0, 32, zero_hist, 0)

        def pass_a(i, _):
            rows3 = (_splat(i * 16) + lane) * 3
            x = plsc.load_gather(coords_v, [rows3])
            y = plsc.load_gather(coords_v, [rows3 + _splat(1)])
            z = plsc.load_gather(coords_v, [rows3 + _splat(2)])
            if shift_origin:
                x = x - ox
                y = y - oy
                z = z - oz
                valid = ((x >= 0) & (x < 96) & (y >= 0) & (y < 96)
                         & (z >= 0) & (z < 96))
                r = (x * 96 + y) * 96 + z
                renc = jnp.where(valid, r, _splat(-1))
            else:
                renc = (x * 96 + y) * 96 + z
                valid = None
            rbuf_v[pl.ds(i * 16, 16)] = renc
            owner = jnp.where(renc >= 0, renc, 0) // SLAB
            col = owner * 16 + lane
            if valid is None:
                plsc.addupdate_scatter(hist_v, [col], _splat(1))
            else:
                plsc.addupdate_scatter(hist_v, [col], _splat(1), mask=valid)
            return 0
        lax.fori_loop(0, nvr, pass_a, 0)

        lane0 = lane == 0

        def prefix(o, base):
            h = hist_v[pl.ds(o * 16, 16)]
            incl = plsc.cumsum(h)
            tot = jnp.sum(h)
            colptr_v[pl.ds(o * 16, 16)] = _splat(base) + (incl - h)
            plsc.store_scatter(counts_v, [_splat(o)], _splat(tot), mask=lane0)
            nbase = base + tot
            return jnp.bitwise_and(nbase + 15, jnp.int32(~15))
        lax.fori_loop(0, 32, prefix, jnp.int32(0))

        def fill(i, _):
            stage_v[pl.ds(i * 16, 16)] = _splat(SENT)
            return 0
        lax.fori_loop(0, BSTAGE // 16, fill, 0)

        def pass_b(i, _):
            renc = rbuf_v[pl.ds(i * 16, 16)]
            valid = renc >= 0
            rr = jnp.where(valid, renc, 0)
            owner = rr // SLAB
            rl = rr - owner * SLAB
            lseq = _splat(i * 16) + lane
            payload = jnp.bitwise_or(lax.shift_left(lseq, _splat(15)), rl)
            col = owner * 16 + lane
            pos = plsc.load_gather(colptr_v, [col])
            plsc.store_scatter(stage_v, [pos], payload, mask=valid)
            plsc.store_scatter(colptr_v, [col], pos + 1, mask=valid)
            return 0
        lax.fori_loop(0, nvr, pass_b, 0)

        pltpu.sync_copy(stage_v.at[pl.ds(0, BSTAGE)],
                        part_hbm.at[pl.ds(pl.multiple_of(w * ROW, 8), BSTAGE)])
        pltpu.sync_copy(
            counts_v,
            counts.at[pl.ds(pl.multiple_of(kind * (NW * 32) + w * 32, 8), 32)])

    run_kind(0, GB, GSTAGE, GROW, glob_coords, part_g, True)
    run_kind(1, CB, CSTAGE, CROW, cur_coords, part_c, False)


@functools.partial(
    pl.kernel,
    mesh=_mesh(),
    compiler_params=_cparams(),
    out_type=jax.ShapeDtypeStruct((V, CH), jnp.float32),
    scratch_types=[
        pltpu.VMEM((2 * NW * 32,), jnp.int32),
        pltpu.VMEM((WTBL,), jnp.int32),
        pltpu.VMEM((SEGCHUNK,), jnp.int32),
        pltpu.VMEM((OCH,), jnp.int32),
        pltpu.VMEM((OCH, CH), jnp.float32),
        pltpu.SemaphoreType.DMA,
    ],
)
def _kernel_b(part_g, part_c, counts, table, out,
              counts_v, winner_v, seg_v, idx_v, rows_v, sem):
    w = lax.axis_index("s") * 2 + lax.axis_index("c")
    lane = _iota()
    pltpu.sync_copy(counts, counts_v)

    def wzero(i, _):
        winner_v[pl.ds(i * 16, 16)] = _splat(-1)
        return 0
    lax.fori_loop(0, WTBL // 16, wzero, 0)

    w16 = _splat(w)

    def ingest_kind(kind, ROW, vr_base_mul, vr_base_add, part_hbm):
        def per_src(src, _):
            b = kind * (NW * 32) + src * 32
            r0 = counts_v[pl.ds(b, 16)]
            r1 = counts_v[pl.ds(b + 16, 16)]
            rnd0 = jnp.bitwise_and(r0 + 15, _splat(~15))
            rnd1 = jnp.bitwise_and(r1 + 15, _splat(~15))
            n = (jnp.sum(jnp.where(lane == w16, r0, 0))
                 + jnp.sum(jnp.where(lane + 16 == w16, r1, 0)))
            off = (jnp.sum(jnp.where(lane < w16, rnd0, 0))
                   + jnp.sum(jnp.where(lane + 16 < w16, rnd1, 0)))
            n16 = jnp.bitwise_and(n + 15, jnp.int32(~15))
            vr_base = src * vr_base_mul + vr_base_add
            nchunks = (n16 + (SEGCHUNK - 1)) // SEGCHUNK

            def per_chunk(c, _):
                pltpu.sync_copy(
                    part_hbm.at[pl.ds(
                        pl.multiple_of(src * ROW + off + c * SEGCHUNK, 8),
                        SEGCHUNK)],
                    seg_v)
                svr = jnp.minimum(SEGCHUNK, n16 - c * SEGCHUNK) // 16

                def per_vreg(j, _):
                    e = seg_v[pl.ds(j * 16, 16)]
                    rl = jnp.bitwise_and(e, _splat(32767))
                    lsq = lax.shift_right_logical(e, _splat(15))
                    key = jnp.bitwise_or(lax.shift_left(rl, _splat(14)), lsq)
                    vr = _splat(vr_base) + lsq
                    sk, sv = plsc.sort_key_val(key, vr)
                    rls = lax.shift_right_logical(sk, _splat(14))
                    old = plsc.load_gather(winner_v, [rls])
                    plsc.store_scatter(winner_v, [rls], jnp.maximum(old, sv))
                    return 0
                lax.fori_loop(0, svr, per_vreg, 0)
                return 0
            lax.fori_loop(0, nchunks, per_chunk, 0)
            return 0
        lax.fori_loop(0, NW, per_src, 0)

    ingest_kind(0, GROW, GB, 0, part_g)
    ingest_kind(1, CROW, CB, NG, part_c)

    def per_out_chunk(c, _):
        def build_idx(v, _):
            wv = winner_v[pl.ds(c * OCH + v * 16, 16)]
            pos = _splat(c * OCH + v * 16) + lane
            dummy = _splat(NG + NC) + jnp.bitwise_and(pos + w16 * 64,
                                                      _splat(NDUMMY - 1))
            idx_v[pl.ds(v * 16, 16)] = jnp.where(wv < 0, dummy, wv)
            return 0
        lax.fori_loop(0, OCH // 16, build_idx, 0)
        pltpu.async_copy(table.at[idx_v], rows_v, sem).wait()
        pltpu.sync_copy(rows_v, out.at[pl.ds(w * SLAB + c * OCH, OCH)])
        return 0
    lax.fori_loop(0, 16, per_out_chunk, 0)


def kernel(current_values, global_values, current_coords, global_coords,
           relative_origin):
    origin_b = jnp.broadcast_to(
        relative_origin.astype(jnp.int32)[:, None], (3, 16)).reshape(48)
    part_g, part_c, counts = _kernel_a(
        current_coords.astype(jnp.int32).reshape(-1),
        global_coords.astype(jnp.int32).reshape(-1), origin_b)
    table = jnp.concatenate(
        [global_values, current_values,
         jnp.zeros((NDUMMY, CH), jnp.float32)], axis=0)
    out = _kernel_b(part_g, part_c, counts, table)
    return out.reshape(96, 96, 96, CH)


# speedup vs baseline: 7.7006x; 1.6296x over previous
import functools

import jax
import jax.numpy as jnp
from jax import lax
from jax.experimental import pallas as pl
from jax.experimental.pallas import tpu as pltpu
from jax.experimental.pallas import tpu_sc as plsc

V = 96 * 96 * 96
NG = 524288
NC = 262144
CH = 16

NW = 32
GB = NG // NW
CB = NC // NW
SLAB = V // NW
WTBL = 32768

GSTAGE = GB + 16 * NW
CSTAGE = CB + 16 * NW
SEGCHUNK = 2048
GROW = GSTAGE + SEGCHUNK
CROW = CSTAGE + SEGCHUNK
OCH = SLAB // 16
NDUMMY = 2048
SENT = 32767

_mesh = lambda: plsc.VectorSubcoreMesh(core_axis_name="c", subcore_axis_name="s")
_cparams = lambda: pltpu.CompilerParams(needs_layout_passes=False,
                                        use_tc_tiling_on_sc=False)


def _iota():
    return lax.iota(jnp.int32, 16)


def _splat(x):
    return jnp.broadcast_to(jnp.asarray(x, jnp.int32), (16,))


@functools.partial(
    pl.kernel,
    mesh=_mesh(),
    compiler_params=_cparams(),
    out_type=(
        jax.ShapeDtypeStruct((NW * GROW,), jnp.int32),
        jax.ShapeDtypeStruct((NW * CROW,), jnp.int32),
        jax.ShapeDtypeStruct((2 * NW * 32,), jnp.int32),
        jax.ShapeDtypeStruct((NG + NC + NDUMMY, CH), jnp.float32),
    ),
    scratch_types=[
        pltpu.VMEM((GB * 3,), jnp.int32),
        pltpu.VMEM((GB,), jnp.int32),
        pltpu.VMEM((GSTAGE,), jnp.int32),
        pltpu.VMEM((512,), jnp.int32),
        pltpu.VMEM((512,), jnp.int32),
        pltpu.VMEM((32,), jnp.int32),
        pltpu.VMEM((48,), jnp.int32),
        pltpu.VMEM((2048, CH), jnp.float32),
    ],
)
def _kernel_a(cur_coords, glob_coords, origin_b, cur_vals, glob_vals,
              part_g, part_c, counts, table,
              coords_v, rbuf_v, stage_v, hist_v, colptr_v, counts_v, origin_v,
              vbuf_v):
    w = lax.axis_index("s") * 2 + lax.axis_index("c")
    lane = _iota()
    zeros16 = _splat(0)
    pltpu.sync_copy(origin_b, origin_v)

    def run_kind(kind, B, BSTAGE, ROW, coords_hbm, part_hbm, shift_origin, N):
        nvr = B // 16
        for c in range(3):
            pltpu.sync_copy(
                coords_hbm.at[pl.ds(pl.multiple_of(c * N + w * B, 8), B)],
                coords_v.at[pl.ds(c * B, B)])

        if shift_origin:
            ox = origin_v[pl.ds(0, 16)]
            oy = origin_v[pl.ds(16, 16)]
            oz = origin_v[pl.ds(32, 16)]

        def zero_hist(i, _):
            hist_v[pl.ds(i * 16, 16)] = zeros16
            return 0
        lax.fori_loop(0, 32, zero_hist, 0)

        def pass_a(i, _):
            x = coords_v[pl.ds(i * 16, 16)]
            y = coords_v[pl.ds(B + i * 16, 16)]
            z = coords_v[pl.ds(2 * B + i * 16, 16)]
            if shift_origin:
                x = x - ox
                y = y - oy
                z = z - oz
                valid = ((x >= 0) & (x < 96) & (y >= 0) & (y < 96)
                         & (z >= 0) & (z < 96))
                r = (x * 96 + y) * 96 + z
                renc = jnp.where(valid, r, _splat(-1))
            else:
                renc = (x * 96 + y) * 96 + z
                valid = None
            rbuf_v[pl.ds(i * 16, 16)] = renc
            owner = jnp.where(renc >= 0, renc, 0) // SLAB
            col = owner * 16 + lane
            if valid is None:
                plsc.addupdate_scatter(hist_v, [col], _splat(1))
            else:
                plsc.addupdate_scatter(hist_v, [col], _splat(1), mask=valid)
            return 0
        lax.fori_loop(0, nvr, pass_a, 0)

        lane0 = lane == 0

        def prefix(o, base):
            h = hist_v[pl.ds(o * 16, 16)]
            incl = plsc.cumsum(h)
            tot = jnp.sum(h)
            colptr_v[pl.ds(o * 16, 16)] = _splat(base) + (incl - h)
            plsc.store_scatter(counts_v, [_splat(o)], _splat(tot), mask=lane0)
            nbase = base + tot
            return jnp.bitwise_and(nbase + 15, jnp.int32(~15))
        lax.fori_loop(0, 32, prefix, jnp.int32(0))

        def fill(i, _):
            stage_v[pl.ds(i * 16, 16)] = _splat(SENT)
            return 0
        lax.fori_loop(0, BSTAGE // 16, fill, 0)

        def pass_b(i, _):
            renc = rbuf_v[pl.ds(i * 16, 16)]
            valid = renc >= 0
            rr = jnp.where(valid, renc, 0)
            owner = rr // SLAB
            rl = rr - owner * SLAB
            lseq = _splat(i * 16) + lane
            payload = jnp.bitwise_or(lax.shift_left(lseq, _splat(15)), rl)
            col = owner * 16 + lane
            pos = plsc.load_gather(colptr_v, [col])
            plsc.store_scatter(stage_v, [pos], payload, mask=valid)
            plsc.store_scatter(colptr_v, [col], pos + 1, mask=valid)
            return 0
        lax.fori_loop(0, nvr, pass_b, 0)

        pltpu.sync_copy(stage_v.at[pl.ds(0, BSTAGE)],
                        part_hbm.at[pl.ds(pl.multiple_of(w * ROW, 8), BSTAGE)])
        pltpu.sync_copy(
            counts_v,
            counts.at[pl.ds(pl.multiple_of(kind * (NW * 32) + w * 32, 8), 32)])

    run_kind(0, GB, GSTAGE, GROW, glob_coords, part_g, True, NG)
    run_kind(1, CB, CSTAGE, CROW, cur_coords, part_c, False, NC)

    def copy_vals(vals_hbm, src_base, dst_base, nchunks):
        def cp(i, _):
            so = pl.multiple_of(src_base + i * 2048, 8)
            do = pl.multiple_of(dst_base + i * 2048, 8)
            pltpu.sync_copy(vals_hbm.at[pl.ds(so, 2048)], vbuf_v)
            pltpu.sync_copy(vbuf_v, table.at[pl.ds(do, 2048)])
            return 0
        lax.fori_loop(0, nchunks, cp, 0)

    copy_vals(glob_vals, w * GB, w * GB, GB // 2048)
    copy_vals(cur_vals, w * CB, NG + w * CB, CB // 2048)

    zrow = jnp.zeros((16,), jnp.float32)

    def zfill(i, _):
        vbuf_v[i, :] = zrow
        return 0
    lax.fori_loop(0, 64, zfill, 0)
    pltpu.sync_copy(
        vbuf_v.at[pl.ds(0, 64)],
        table.at[pl.ds(pl.multiple_of(NG + NC + w * 64, 8), 64)])


@functools.partial(
    pl.kernel,
    mesh=_mesh(),
    compiler_params=_cparams(),
    out_type=jax.ShapeDtypeStruct((V, CH), jnp.float32),
    scratch_types=[
        pltpu.VMEM((2 * NW * 32,), jnp.int32),
        pltpu.VMEM((WTBL,), jnp.int32),
        pltpu.VMEM((SEGCHUNK,), jnp.int32),
        pltpu.VMEM((OCH,), jnp.int32),
        pltpu.VMEM((OCH, CH), jnp.float32),
        pltpu.SemaphoreType.DMA,
    ],
)
def _kernel_b(part_g, part_c, counts, table, out,
              counts_v, winner_v, seg_v, idx_v, rows_v, sem):
    w = lax.axis_index("s") * 2 + lax.axis_index("c")
    lane = _iota()
    pltpu.sync_copy(counts, counts_v)

    def wzero(i, _):
        winner_v[pl.ds(i * 16, 16)] = _splat(-1)
        return 0
    lax.fori_loop(0, WTBL // 16, wzero, 0)

    w16 = _splat(w)

    def ingest_kind(kind, ROW, vr_base_mul, vr_base_add, part_hbm):
        def per_src(src, _):
            b = kind * (NW * 32) + src * 32
            r0 = counts_v[pl.ds(b, 16)]
            r1 = counts_v[pl.ds(b + 16, 16)]
            rnd0 = jnp.bitwise_and(r0 + 15, _splat(~15))
            rnd1 = jnp.bitwise_and(r1 + 15, _splat(~15))
            n = (jnp.sum(jnp.where(lane == w16, r0, 0))
                 + jnp.sum(jnp.where(lane + 16 == w16, r1, 0)))
            off = (jnp.sum(jnp.where(lane < w16, rnd0, 0))
                   + jnp.sum(jnp.where(lane + 16 < w16, rnd1, 0)))
            n16 = jnp.bitwise_and(n + 15, jnp.int32(~15))
            vr_base = src * vr_base_mul + vr_base_add
            nchunks = (n16 + (SEGCHUNK - 1)) // SEGCHUNK

            def per_chunk(c, _):
                pltpu.sync_copy(
                    part_hbm.at[pl.ds(
                        pl.multiple_of(src * ROW + off + c * SEGCHUNK, 8),
                        SEGCHUNK)],
                    seg_v)
                svr = jnp.minimum(SEGCHUNK, n16 - c * SEGCHUNK) // 16

                def per_vreg(j, _):
                    e = seg_v[pl.ds(j * 16, 16)]
                    rl = jnp.bitwise_and(e, _splat(32767))
                    lsq = lax.shift_right_logical(e, _splat(15))
                    key = jnp.bitwise_or(lax.shift_left(rl, _splat(14)), lsq)
                    vr = _splat(vr_base) + lsq
                    sk, sv = plsc.sort_key_val(key, vr)
                    rls = lax.shift_right_logical(sk, _splat(14))
                    old = plsc.load_gather(winner_v, [rls])
                    plsc.store_scatter(winner_v, [rls], jnp.maximum(old, sv))
                    return 0
                lax.fori_loop(0, svr, per_vreg, 0)
                return 0
            lax.fori_loop(0, nchunks, per_chunk, 0)
            return 0
        lax.fori_loop(0, NW, per_src, 0)

    ingest_kind(0, GROW, GB, 0, part_g)
    ingest_kind(1, CROW, CB, NG, part_c)

    def per_out_chunk(c, _):
        def build_idx(v, _):
            wv = winner_v[pl.ds(c * OCH + v * 16, 16)]
            pos = _splat(c * OCH + v * 16) + lane
            dummy = _splat(NG + NC) + jnp.bitwise_and(pos + w16 * 64,
                                                      _splat(NDUMMY - 1))
            idx_v[pl.ds(v * 16, 16)] = jnp.where(wv < 0, dummy, wv)
            return 0
        lax.fori_loop(0, OCH // 16, build_idx, 0)
        pltpu.async_copy(table.at[idx_v], rows_v, sem).wait()
        pltpu.sync_copy(rows_v, out.at[pl.ds(w * SLAB + c * OCH, OCH)])
        return 0
    lax.fori_loop(0, 16, per_out_chunk, 0)


def kernel(current_values, global_values, current_coords, global_coords,
           relative_origin):
    origin_b = jnp.broadcast_to(
        relative_origin.astype(jnp.int32)[:, None], (3, 16)).reshape(48)
    part_g, part_c, counts, table = _kernel_a(
        current_coords.astype(jnp.int32).T.reshape(-1),
        global_coords.astype(jnp.int32).T.reshape(-1), origin_b,
        current_values, global_values)
    out = _kernel_b(part_g, part_c, counts, table)
    return out.reshape(96, 96, 96, CH)


# speedup vs baseline: 8.6639x; 1.1251x over previous
import functools

import jax
import jax.numpy as jnp
from jax import lax
from jax.experimental import pallas as pl
from jax.experimental.pallas import tpu as pltpu
from jax.experimental.pallas import tpu_sc as plsc

V = 96 * 96 * 96
NG = 524288
NC = 262144
CH = 16

NW = 32
GB = NG // NW
CB = NC // NW
SLAB = V // NW
WTBL = 32768

GSTAGE = GB + 16 * NW
CSTAGE = CB + 16 * NW
SEGCHUNK = 2048
GROW = GSTAGE + SEGCHUNK
CROW = CSTAGE + SEGCHUNK
PCH = 16
CHN = PCH * 96
NDUMMY = 2048
SENT = 32767

_mesh = lambda: plsc.VectorSubcoreMesh(core_axis_name="c", subcore_axis_name="s")
_cparams = lambda: pltpu.CompilerParams(needs_layout_passes=False,
                                        use_tc_tiling_on_sc=False)


def _iota():
    return lax.iota(jnp.int32, 16)


def _splat(x):
    return jnp.broadcast_to(jnp.asarray(x, jnp.int32), (16,))


@functools.partial(
    pl.kernel,
    mesh=_mesh(),
    compiler_params=_cparams(),
    out_type=(
        jax.ShapeDtypeStruct((NW * GROW,), jnp.int32),
        jax.ShapeDtypeStruct((NW * CROW,), jnp.int32),
        jax.ShapeDtypeStruct((2 * NW * 32,), jnp.int32),
        jax.ShapeDtypeStruct((NG + NC + NDUMMY, CH), jnp.float32),
    ),
    scratch_types=[
        pltpu.VMEM((GB * 3,), jnp.int32),
        pltpu.VMEM((GB,), jnp.int32),
        pltpu.VMEM((GSTAGE,), jnp.int32),
        pltpu.VMEM((512,), jnp.int32),
        pltpu.VMEM((512,), jnp.int32),
        pltpu.VMEM((32,), jnp.int32),
        pltpu.VMEM((48,), jnp.int32),
        pltpu.VMEM((2048, CH), jnp.float32),
    ],
)
def _kernel_a(cur_coords, glob_coords, origin_b, cur_vals, glob_vals,
              part_g, part_c, counts, table,
              coords_v, rbuf_v, stage_v, hist_v, colptr_v, counts_v, origin_v,
              vbuf_v):
    w = lax.axis_index("s") * 2 + lax.axis_index("c")
    lane = _iota()
    zeros16 = _splat(0)
    pltpu.sync_copy(origin_b, origin_v)

    def run_kind(kind, B, BSTAGE, ROW, coords_hbm, part_hbm, shift_origin, N):
        nvr = B // 16
        for c in range(3):
            pltpu.sync_copy(
                coords_hbm.at[pl.ds(pl.multiple_of(c * N + w * B, 8), B)],
                coords_v.at[pl.ds(c * B, B)])

        if shift_origin:
            ox = origin_v[pl.ds(0, 16)]
            oy = origin_v[pl.ds(16, 16)]
            oz = origin_v[pl.ds(32, 16)]

        def zero_hist(i, _):
            hist_v[pl.ds(i * 16, 16)] = zeros16
            return 0
        lax.fori_loop(0, 32, zero_hist, 0)

        def pass_a(i, _):
            x = coords_v[pl.ds(i * 16, 16)]
            y = coords_v[pl.ds(B + i * 16, 16)]
            z = coords_v[pl.ds(2 * B + i * 16, 16)]
            if shift_origin:
                x = x - ox
                y = y - oy
                z = z - oz
                valid = ((x >= 0) & (x < 96) & (y >= 0) & (y < 96)
                         & (z >= 0) & (z < 96))
                r = (x * 96 + y) * 96 + z
                renc = jnp.where(valid, r, _splat(-1))
            else:
                renc = (x * 96 + y) * 96 + z
                valid = None
            rbuf_v[pl.ds(i * 16, 16)] = renc
            owner = jnp.where(renc >= 0, renc, 0) // SLAB
            col = owner * 16 + lane
            if valid is None:
                plsc.addupdate_scatter(hist_v, [col], _splat(1))
            else:
                plsc.addupdate_scatter(hist_v, [col], _splat(1), mask=valid)
            return 0
        lax.fori_loop(0, nvr, pass_a, 0)

        lane0 = lane == 0

        def prefix(o, base):
            h = hist_v[pl.ds(o * 16, 16)]
            incl = plsc.cumsum(h)
            tot = jnp.sum(h)
            colptr_v[pl.ds(o * 16, 16)] = _splat(base) + (incl - h)
            plsc.store_scatter(counts_v, [_splat(o)], _splat(tot), mask=lane0)
            nbase = base + tot
            return jnp.bitwise_and(nbase + 15, jnp.int32(~15))
        lax.fori_loop(0, 32, prefix, jnp.int32(0))

        def fill(i, _):
            stage_v[pl.ds(i * 16, 16)] = _splat(SENT)
            return 0
        lax.fori_loop(0, BSTAGE // 16, fill, 0)

        def pass_b(i, _):
            renc = rbuf_v[pl.ds(i * 16, 16)]
            valid = renc >= 0
            rr = jnp.where(valid, renc, 0)
            owner = rr // SLAB
            rl = rr - owner * SLAB
            lseq = _splat(i * 16) + lane
            payload = jnp.bitwise_or(lax.shift_left(lseq, _splat(15)), rl)
            col = owner * 16 + lane
            pos = plsc.load_gather(colptr_v, [col])
            plsc.store_scatter(stage_v, [pos], payload, mask=valid)
            plsc.store_scatter(colptr_v, [col], pos + 1, mask=valid)
            return 0
        lax.fori_loop(0, nvr, pass_b, 0)

        pltpu.sync_copy(stage_v.at[pl.ds(0, BSTAGE)],
                        part_hbm.at[pl.ds(pl.multiple_of(w * ROW, 8), BSTAGE)])
        pltpu.sync_copy(
            counts_v,
            counts.at[pl.ds(pl.multiple_of(kind * (NW * 32) + w * 32, 8), 32)])

    run_kind(0, GB, GSTAGE, GROW, glob_coords, part_g, True, NG)
    run_kind(1, CB, CSTAGE, CROW, cur_coords, part_c, False, NC)

    def copy_vals(vals_hbm, src_base, dst_base, nchunks):
        def cp(i, _):
            so = pl.multiple_of(src_base + i * 2048, 8)
            do = pl.multiple_of(dst_base + i * 2048, 8)
            pltpu.sync_copy(vals_hbm.at[pl.ds(so, 2048)], vbuf_v)
            pltpu.sync_copy(vbuf_v, table.at[pl.ds(do, 2048)])
            return 0
        lax.fori_loop(0, nchunks, cp, 0)

    copy_vals(glob_vals, w * GB, w * GB, GB // 2048)
    copy_vals(cur_vals, w * CB, NG + w * CB, CB // 2048)

    zrow = jnp.zeros((16,), jnp.float32)

    def zfill(i, _):
        vbuf_v[i, :] = zrow
        return 0
    lax.fori_loop(0, 64, zfill, 0)
    pltpu.sync_copy(
        vbuf_v.at[pl.ds(0, 64)],
        table.at[pl.ds(pl.multiple_of(NG + NC + w * 64, 8), 64)])


@functools.partial(
    pl.kernel,
    mesh=_mesh(),
    compiler_params=_cparams(),
    out_type=jax.ShapeDtypeStruct((96, 96, 2, 8, 128), jnp.float32),
    scratch_types=[
        pltpu.VMEM((2 * NW * 32,), jnp.int32),
        pltpu.VMEM((WTBL,), jnp.int32),
        pltpu.VMEM((SEGCHUNK,), jnp.int32),
        pltpu.VMEM((CHN,), jnp.int32),
        pltpu.VMEM((CHN, CH), jnp.float32),
        pltpu.VMEM((PCH, 2, 8, 128), jnp.float32),
        pltpu.SemaphoreType.DMA,
    ],
)
def _kernel_b(part_g, part_c, counts, table, out,
              counts_v, winner_v, seg_v, idx_v, rows_v, pen_v, sem):
    w = lax.axis_index("s") * 2 + lax.axis_index("c")
    lane = _iota()
    pltpu.sync_copy(counts, counts_v)

    def wzero(i, _):
        winner_v[pl.ds(i * 16, 16)] = _splat(-1)
        return 0
    lax.fori_loop(0, WTBL // 16, wzero, 0)

    w16 = _splat(w)

    def ingest_kind(kind, ROW, vr_base_mul, vr_base_add, part_hbm):
        def per_src(src, _):
            b = kind * (NW * 32) + src * 32
            r0 = counts_v[pl.ds(b, 16)]
            r1 = counts_v[pl.ds(b + 16, 16)]
            rnd0 = jnp.bitwise_and(r0 + 15, _splat(~15))
            rnd1 = jnp.bitwise_and(r1 + 15, _splat(~15))
            n = (jnp.sum(jnp.where(lane == w16, r0, 0))
                 + jnp.sum(jnp.where(lane + 16 == w16, r1, 0)))
            off = (jnp.sum(jnp.where(lane < w16, rnd0, 0))
                   + jnp.sum(jnp.where(lane + 16 < w16, rnd1, 0)))
            n16 = jnp.bitwise_and(n + 15, jnp.int32(~15))
            vr_base = src * vr_base_mul + vr_base_add
            nchunks = (n16 + (SEGCHUNK - 1)) // SEGCHUNK

            def per_chunk(c, _):
                pltpu.sync_copy(
                    part_hbm.at[pl.ds(
                        pl.multiple_of(src * ROW + off + c * SEGCHUNK, 8),
                        SEGCHUNK)],
                    seg_v)
                svr = jnp.minimum(SEGCHUNK, n16 - c * SEGCHUNK) // 16

                def per_vreg(j, _):
                    e = seg_v[pl.ds(j * 16, 16)]
                    rl = jnp.bitwise_and(e, _splat(32767))
                    lsq = lax.shift_right_logical(e, _splat(15))
                    key = jnp.bitwise_or(lax.shift_left(rl, _splat(14)), lsq)
                    vr = _splat(vr_base) + lsq
                    sk, sv = plsc.sort_key_val(key, vr)
                    rls = lax.shift_right_logical(sk, _splat(14))
                    old = plsc.load_gather(winner_v, [rls])
                    plsc.store_scatter(winner_v, [rls], jnp.maximum(old, sv))
                    return 0
                lax.fori_loop(0, svr, per_vreg, 0)
                return 0
            lax.fori_loop(0, nchunks, per_chunk, 0)
            return 0
        lax.fori_loop(0, NW, per_src, 0)

    ingest_kind(0, GROW, GB, 0, part_g)
    ingest_kind(1, CROW, CB, NG, part_c)

    c2l = lax.shift_right_logical(lane, _splat(3))
    c8l = jnp.bitwise_and(lane, _splat(7))

    def per_out_chunk(c, _):
        def build_idx(v, _):
            wv = winner_v[pl.ds(c * CHN + v * 16, 16)]
            pos = _splat(c * CHN + v * 16) + lane
            dummy = _splat(NG + NC) + jnp.bitwise_and(pos + w16 * 64,
                                                      _splat(NDUMMY - 1))
            idx_v[pl.ds(v * 16, 16)] = jnp.where(wv < 0, dummy, wv)
            return 0
        lax.fori_loop(0, CHN // 16, build_idx, 0)
        pltpu.async_copy(table.at[idx_v], rows_v, sem).wait()

        def xpose_p(p, _):
            def xpose_z(z, _):
                row = rows_v[p * 96 + z, :]
                plsc.store_scatter(
                    pen_v, [_splat(p), c2l, c8l, _splat(z)], row)
                return 0
            lax.fori_loop(0, 96, xpose_z, 0)
            return 0
        lax.fori_loop(0, PCH, xpose_p, 0)

        x0 = w * 3 + c // 6
        y0 = (c % 6) * PCH
        pltpu.sync_copy(pen_v, out.at[x0, pl.ds(y0, PCH)])
        return 0
    lax.fori_loop(0, 18, per_out_chunk, 0)


def kernel(current_values, global_values, current_coords, global_coords,
           relative_origin):
    origin_b = jnp.broadcast_to(
        relative_origin.astype(jnp.int32)[:, None], (3, 16)).reshape(48)
    part_g, part_c, counts, table = _kernel_a(
        current_coords.astype(jnp.int32).T.reshape(-1),
        global_coords.astype(jnp.int32).T.reshape(-1), origin_b,
        current_values, global_values)
    out5 = _kernel_b(part_g, part_c, counts, table)
    return out5[:, :, :, :, :96].transpose(0, 1, 4, 2, 3).reshape(
        96, 96, 96, CH)


# speedup vs baseline: 8.6872x; 1.0027x over previous
import functools

import jax
import jax.numpy as jnp
from jax import lax
from jax.experimental import pallas as pl
from jax.experimental.pallas import tpu as pltpu
from jax.experimental.pallas import tpu_sc as plsc

V = 96 * 96 * 96
NG = 524288
NC = 262144
CH = 16

NW = 32
GB = NG // NW
CB = NC // NW
SLAB = V // NW
WTBL = 32768

GSTAGE = GB + 16 * NW
CSTAGE = CB + 16 * NW
SEGCHUNK = 2048
GROW = GSTAGE + SEGCHUNK
CROW = CSTAGE + SEGCHUNK
PCH = 16
CHN = PCH * 96
NDUMMY = 2048
SENT = 32767

_mesh = lambda: plsc.VectorSubcoreMesh(core_axis_name="c", subcore_axis_name="s")
_cparams = lambda: pltpu.CompilerParams(needs_layout_passes=False,
                                        use_tc_tiling_on_sc=False)


def _iota():
    return lax.iota(jnp.int32, 16)


def _splat(x):
    return jnp.broadcast_to(jnp.asarray(x, jnp.int32), (16,))


@functools.partial(
    pl.kernel,
    mesh=_mesh(),
    compiler_params=_cparams(),
    out_type=(
        jax.ShapeDtypeStruct((NW * GROW,), jnp.int32),
        jax.ShapeDtypeStruct((NW * CROW,), jnp.int32),
        jax.ShapeDtypeStruct((2 * NW * 32,), jnp.int32),
        jax.ShapeDtypeStruct((NG + NC + NDUMMY, CH), jnp.float32),
    ),
    scratch_types=[
        pltpu.VMEM((GB * 3,), jnp.int32),
        pltpu.VMEM((GB,), jnp.int32),
        pltpu.VMEM((GSTAGE,), jnp.int32),
        pltpu.VMEM((512,), jnp.int32),
        pltpu.VMEM((512,), jnp.int32),
        pltpu.VMEM((32,), jnp.int32),
        pltpu.VMEM((48,), jnp.int32),
        pltpu.VMEM((2048, CH), jnp.float32),
    ],
)
def _kernel_a(cur_coords, glob_coords, origin_b, cur_vals, glob_vals,
              part_g, part_c, counts, table,
              coords_v, rbuf_v, stage_v, hist_v, colptr_v, counts_v, origin_v,
              vbuf_v):
    w = lax.axis_index("s") * 2 + lax.axis_index("c")
    lane = _iota()
    zeros16 = _splat(0)
    pltpu.sync_copy(origin_b, origin_v)

    def run_kind(kind, B, BSTAGE, ROW, coords_hbm, part_hbm, shift_origin, N):
        nvr = B // 16
        for c in range(3):
            pltpu.sync_copy(
                coords_hbm.at[pl.ds(pl.multiple_of(c * N + w * B, 8), B)],
                coords_v.at[pl.ds(c * B, B)])

        if shift_origin:
            ox = origin_v[pl.ds(0, 16)]
            oy = origin_v[pl.ds(16, 16)]
            oz = origin_v[pl.ds(32, 16)]

        def zero_hist(i, _):
            hist_v[pl.ds(i * 16, 16)] = zeros16
            return 0
        lax.fori_loop(0, 32, zero_hist, 0)

        def pass_a(i, _):
            x = coords_v[pl.ds(i * 16, 16)]
            y = coords_v[pl.ds(B + i * 16, 16)]
            z = coords_v[pl.ds(2 * B + i * 16, 16)]
            if shift_origin:
                x = x - ox
                y = y - oy
                z = z - oz
                valid = ((x >= 0) & (x < 96) & (y >= 0) & (y < 96)
                         & (z >= 0) & (z < 96))
                r = (x * 96 + y) * 96 + z
                renc = jnp.where(valid, r, _splat(-1))
            else:
                renc = (x * 96 + y) * 96 + z
                valid = None
            rbuf_v[pl.ds(i * 16, 16)] = renc
            owner = jnp.where(renc >= 0, renc, 0) // SLAB
            col = owner * 16 + lane
            if valid is None:
                plsc.addupdate_scatter(hist_v, [col], _splat(1))
            else:
                plsc.addupdate_scatter(hist_v, [col], _splat(1), mask=valid)
            return 0
        lax.fori_loop(0, nvr, pass_a, 0)

        lane0 = lane == 0

        def prefix(o, base):
            h = hist_v[pl.ds(o * 16, 16)]
            incl = plsc.cumsum(h)
            tot = jnp.sum(h)
            colptr_v[pl.ds(o * 16, 16)] = _splat(base) + (incl - h)
            plsc.store_scatter(counts_v, [_splat(o)], _splat(tot), mask=lane0)
            nbase = base + tot
            return jnp.bitwise_and(nbase + 15, jnp.int32(~15))
        lax.fori_loop(0, 32, prefix, jnp.int32(0))

        def fill(i, _):
            stage_v[pl.ds(i * 16, 16)] = _splat(SENT)
            return 0
        lax.fori_loop(0, BSTAGE // 16, fill, 0)

        def pass_b(i, _):
            renc = rbuf_v[pl.ds(i * 16, 16)]
            valid = renc >= 0
            rr = jnp.where(valid, renc, 0)
            owner = rr // SLAB
            rl = rr - owner * SLAB
            lseq = _splat(i * 16) + lane
            payload = jnp.bitwise_or(lax.shift_left(lseq, _splat(15)), rl)
            col = owner * 16 + lane
            pos = plsc.load_gather(colptr_v, [col])
            plsc.store_scatter(stage_v, [pos], payload, mask=valid)
            plsc.store_scatter(colptr_v, [col], pos + 1, mask=valid)
            return 0
        lax.fori_loop(0, nvr, pass_b, 0)

        pltpu.sync_copy(stage_v.at[pl.ds(0, BSTAGE)],
                        part_hbm.at[pl.ds(pl.multiple_of(w * ROW, 8), BSTAGE)])
        pltpu.sync_copy(
            counts_v,
            counts.at[pl.ds(pl.multiple_of(kind * (NW * 32) + w * 32, 8), 32)])

    run_kind(0, GB, GSTAGE, GROW, glob_coords, part_g, True, NG)
    run_kind(1, CB, CSTAGE, CROW, cur_coords, part_c, False, NC)

    def copy_vals(vals_hbm, src_base, dst_base, nchunks):
        def cp(i, _):
            so = pl.multiple_of(src_base + i * 2048, 8)
            do = pl.multiple_of(dst_base + i * 2048, 8)
            pltpu.sync_copy(vals_hbm.at[pl.ds(so, 2048)], vbuf_v)
            pltpu.sync_copy(vbuf_v, table.at[pl.ds(do, 2048)])
            return 0
        lax.fori_loop(0, nchunks, cp, 0)

    copy_vals(glob_vals, w * GB, w * GB, GB // 2048)
    copy_vals(cur_vals, w * CB, NG + w * CB, CB // 2048)

    zrow = jnp.zeros((16,), jnp.float32)

    def zfill(i, _):
        vbuf_v[i, :] = zrow
        return 0
    lax.fori_loop(0, 64, zfill, 0)
    pltpu.sync_copy(
        vbuf_v.at[pl.ds(0, 64)],
        table.at[pl.ds(pl.multiple_of(NG + NC + w * 64, 8), 64)])


@functools.partial(
    pl.kernel,
    mesh=_mesh(),
    compiler_params=_cparams(),
    out_type=jax.ShapeDtypeStruct((96, 96, 2, 8, 128), jnp.float32),
    scratch_types=[
        pltpu.VMEM((2 * NW * 32,), jnp.int32),
        pltpu.VMEM((WTBL,), jnp.int32),
        pltpu.VMEM((SEGCHUNK,), jnp.int32),
        pltpu.VMEM((CHN,), jnp.int32),
        pltpu.VMEM((CHN, CH), jnp.float32),
        pltpu.VMEM((PCH, 2, 8, 128), jnp.float32),
        pltpu.SemaphoreType.DMA,
    ],
)
def _kernel_b(part_g, part_c, counts, table, out,
              counts_v, winner_v, seg_v, idx_v, rows_v, pen_v, sem):
    w = lax.axis_index("s") * 2 + lax.axis_index("c")
    lane = _iota()
    pltpu.sync_copy(counts, counts_v)

    def wzero(i, _):
        winner_v[pl.ds(i * 16, 16)] = _splat(-1)
        return 0
    lax.fori_loop(0, WTBL // 16, wzero, 0)

    w16 = _splat(w)

    def ingest_kind(kind, ROW, vr_base_mul, vr_base_add, part_hbm):
        def per_src(src, _):
            b = kind * (NW * 32) + src * 32
            r0 = counts_v[pl.ds(b, 16)]
            r1 = counts_v[pl.ds(b + 16, 16)]
            rnd0 = jnp.bitwise_and(r0 + 15, _splat(~15))
            rnd1 = jnp.bitwise_and(r1 + 15, _splat(~15))
            n = (jnp.sum(jnp.where(lane == w16, r0, 0))
                 + jnp.sum(jnp.where(lane + 16 == w16, r1, 0)))
            off = (jnp.sum(jnp.where(lane < w16, rnd0, 0))
                   + jnp.sum(jnp.where(lane + 16 < w16, rnd1, 0)))
            n16 = jnp.bitwise_and(n + 15, jnp.int32(~15))
            vr_base = src * vr_base_mul + vr_base_add
            nchunks = (n16 + (SEGCHUNK - 1)) // SEGCHUNK

            def per_chunk(c, _):
                pltpu.sync_copy(
                    part_hbm.at[pl.ds(
                        pl.multiple_of(src * ROW + off + c * SEGCHUNK, 8),
                        SEGCHUNK)],
                    seg_v)
                svr = jnp.minimum(SEGCHUNK, n16 - c * SEGCHUNK) // 16

                def per_vreg(j, _):
                    e = seg_v[pl.ds(j * 16, 16)]
                    rl = jnp.bitwise_and(e, _splat(32767))
                    lsq = lax.shift_right_logical(e, _splat(15))
                    key = jnp.bitwise_or(lax.shift_left(rl, _splat(14)), lsq)
                    vr = _splat(vr_base) + lsq
                    sk, sv = plsc.sort_key_val(key, vr)
                    rls = lax.shift_right_logical(sk, _splat(14))
                    old = plsc.load_gather(winner_v, [rls])
                    plsc.store_scatter(winner_v, [rls], jnp.maximum(old, sv))
                    return 0
                lax.fori_loop(0, svr, per_vreg, 0)
                return 0
            lax.fori_loop(0, nchunks, per_chunk, 0)
            return 0
        lax.fori_loop(0, NW, per_src, 0)

    ingest_kind(0, GROW, GB, 0, part_g)
    ingest_kind(1, CROW, CB, NG, part_c)

    c2l = lax.shift_right_logical(lane, _splat(3))
    c8l = jnp.bitwise_and(lane, _splat(7))

    def per_out_chunk(c, _):
        def build_idx(v, _):
            wv = winner_v[pl.ds(c * CHN + v * 16, 16)]
            pos = _splat(c * CHN + v * 16) + lane
            dummy = _splat(NG + NC) + jnp.bitwise_and(pos + w16 * 64,
                                                      _splat(NDUMMY - 1))
            idx_v[pl.ds(v * 16, 16)] = jnp.where(wv < 0, dummy, wv)
            return 0
        lax.fori_loop(0, CHN // 16, build_idx, 0)
        pltpu.async_copy(table.at[idx_v], rows_v, sem).wait()

        def xpose_p(p, _):
            def xpose_z(zg, _):
                z0 = zg * 8
                for u in range(8):
                    row = rows_v[p * 96 + z0 + u, :]
                    plsc.store_scatter(
                        pen_v, [_splat(p), c2l, c8l, _splat(z0 + u)], row)
                return 0
            lax.fori_loop(0, 12, xpose_z, 0)
            return 0
        lax.fori_loop(0, PCH, xpose_p, 0)

        x0 = w * 3 + c // 6
        y0 = (c % 6) * PCH
        pltpu.sync_copy(pen_v, out.at[x0, pl.ds(y0, PCH)])
        return 0
    lax.fori_loop(0, 18, per_out_chunk, 0)


def kernel(current_values, global_values, current_coords, global_coords,
           relative_origin):
    origin_b = jnp.broadcast_to(
        relative_origin.astype(jnp.int32)[:, None], (3, 16)).reshape(48)
    part_g, part_c, counts, table = _kernel_a(
        current_coords.astype(jnp.int32).T.reshape(-1),
        global_coords.astype(jnp.int32).T.reshape(-1), origin_b,
        current_values, global_values)
    out5 = _kernel_b(part_g, part_c, counts, table)
    return out5[:, :, :, :, :96].transpose(0, 1, 4, 2, 3).reshape(
        96, 96, 96, CH)


# speedup vs baseline: 9.6983x; 1.1164x over previous
import functools

import jax
import jax.numpy as jnp
from jax import lax
from jax.experimental import pallas as pl
from jax.experimental.pallas import tpu as pltpu
from jax.experimental.pallas import tpu_sc as plsc

V = 96 * 96 * 96
NG = 524288
NC = 262144
CH = 16

NW = 32
GB = NG // NW
CB = NC // NW
SLAB = V // NW
WTBL = 32768

GSTAGE = GB + 16 * NW
CSTAGE = CB + 16 * NW
SEGCHUNK = 2048
GROW = GSTAGE + SEGCHUNK
CROW = CSTAGE + SEGCHUNK
PCH = 16
CHN = PCH * 96
NDUMMY = 2048
SENT = 32767

_mesh = lambda: plsc.VectorSubcoreMesh(core_axis_name="c", subcore_axis_name="s")
_cparams = lambda: pltpu.CompilerParams(needs_layout_passes=False,
                                        use_tc_tiling_on_sc=False)


def _iota():
    return lax.iota(jnp.int32, 16)


def _splat(x):
    return jnp.broadcast_to(jnp.asarray(x, jnp.int32), (16,))


@functools.partial(
    pl.kernel,
    mesh=_mesh(),
    compiler_params=_cparams(),
    out_type=(
        jax.ShapeDtypeStruct((NW * GROW,), jnp.int32),
        jax.ShapeDtypeStruct((NW * CROW,), jnp.int32),
        jax.ShapeDtypeStruct((2 * NW * 32,), jnp.int32),
        jax.ShapeDtypeStruct((NG + NC + NDUMMY, CH), jnp.float32),
    ),
    scratch_types=[
        pltpu.VMEM((GB * 3,), jnp.int32),
        pltpu.VMEM((GB,), jnp.int32),
        pltpu.VMEM((GSTAGE,), jnp.int32),
        pltpu.VMEM((512,), jnp.int32),
        pltpu.VMEM((512,), jnp.int32),
        pltpu.VMEM((32,), jnp.int32),
        pltpu.VMEM((48,), jnp.int32),
        pltpu.VMEM((2048, CH), jnp.float32),
    ],
)
def _kernel_a(cur_coords, glob_coords, origin_b, cur_vals, glob_vals,
              part_g, part_c, counts, table,
              coords_v, rbuf_v, stage_v, hist_v, colptr_v, counts_v, origin_v,
              vbuf_v):
    w = lax.axis_index("s") * 2 + lax.axis_index("c")
    lane = _iota()
    zeros16 = _splat(0)
    pltpu.sync_copy(origin_b, origin_v)

    def run_kind(kind, B, BSTAGE, ROW, coords_hbm, part_hbm, shift_origin, N):
        nvr = B // 16
        for c in range(3):
            pltpu.sync_copy(
                coords_hbm.at[pl.ds(pl.multiple_of(c * N + w * B, 8), B)],
                coords_v.at[pl.ds(c * B, B)])

        if shift_origin:
            ox = origin_v[pl.ds(0, 16)]
            oy = origin_v[pl.ds(16, 16)]
            oz = origin_v[pl.ds(32, 16)]

        def zero_hist(i, _):
            hist_v[pl.ds(i * 16, 16)] = zeros16
            return 0
        lax.fori_loop(0, 32, zero_hist, 0)

        def pass_a(i, _):
            x = coords_v[pl.ds(i * 16, 16)]
            y = coords_v[pl.ds(B + i * 16, 16)]
            z = coords_v[pl.ds(2 * B + i * 16, 16)]
            if shift_origin:
                x = x - ox
                y = y - oy
                z = z - oz
                valid = ((x >= 0) & (x < 96) & (y >= 0) & (y < 96)
                         & (z >= 0) & (z < 96))
                r = (x * 96 + y) * 96 + z
                renc = jnp.where(valid, r, _splat(-1))
            else:
                renc = (x * 96 + y) * 96 + z
                valid = None
            rbuf_v[pl.ds(i * 16, 16)] = renc
            owner = jnp.where(renc >= 0, renc, 0) // SLAB
            col = owner * 16 + lane
            if valid is None:
                plsc.addupdate_scatter(hist_v, [col], _splat(1))
            else:
                plsc.addupdate_scatter(hist_v, [col], _splat(1), mask=valid)
            return 0
        lax.fori_loop(0, nvr, pass_a, 0)

        lane0 = lane == 0

        def prefix(o, base):
            h = hist_v[pl.ds(o * 16, 16)]
            incl = plsc.cumsum(h)
            tot = jnp.sum(h)
            colptr_v[pl.ds(o * 16, 16)] = _splat(base) + (incl - h)
            plsc.store_scatter(counts_v, [_splat(o)], _splat(tot), mask=lane0)
            nbase = base + tot
            return jnp.bitwise_and(nbase + 15, jnp.int32(~15))
        lax.fori_loop(0, 32, prefix, jnp.int32(0))

        def fill(i, _):
            stage_v[pl.ds(i * 16, 16)] = _splat(SENT)
            return 0
        lax.fori_loop(0, BSTAGE // 16, fill, 0)

        def pass_b(i, _):
            renc = rbuf_v[pl.ds(i * 16, 16)]
            valid = renc >= 0
            rr = jnp.where(valid, renc, 0)
            owner = rr // SLAB
            rl = rr - owner * SLAB
            lseq = _splat(i * 16) + lane
            payload = jnp.bitwise_or(lax.shift_left(lseq, _splat(15)), rl)
            col = owner * 16 + lane
            pos = plsc.load_gather(colptr_v, [col])
            plsc.store_scatter(stage_v, [pos], payload, mask=valid)
            plsc.store_scatter(colptr_v, [col], pos + 1, mask=valid)
            return 0
        lax.fori_loop(0, nvr, pass_b, 0)

        pltpu.sync_copy(stage_v.at[pl.ds(0, BSTAGE)],
                        part_hbm.at[pl.ds(pl.multiple_of(w * ROW, 8), BSTAGE)])
        pltpu.sync_copy(
            counts_v,
            counts.at[pl.ds(pl.multiple_of(kind * (NW * 32) + w * 32, 8), 32)])

    run_kind(0, GB, GSTAGE, GROW, glob_coords, part_g, True, NG)
    run_kind(1, CB, CSTAGE, CROW, cur_coords, part_c, False, NC)

    def copy_vals(vals_hbm, src_base, dst_base, nchunks):
        def cp(i, _):
            so = pl.multiple_of(src_base + i * 2048, 8)
            do = pl.multiple_of(dst_base + i * 2048, 8)
            pltpu.sync_copy(vals_hbm.at[pl.ds(so, 2048)], vbuf_v)
            pltpu.sync_copy(vbuf_v, table.at[pl.ds(do, 2048)])
            return 0
        lax.fori_loop(0, nchunks, cp, 0)

    copy_vals(glob_vals, w * GB, w * GB, GB // 2048)
    copy_vals(cur_vals, w * CB, NG + w * CB, CB // 2048)

    zrow = jnp.zeros((16,), jnp.float32)

    def zfill(i, _):
        vbuf_v[i, :] = zrow
        return 0
    lax.fori_loop(0, 64, zfill, 0)
    pltpu.sync_copy(
        vbuf_v.at[pl.ds(0, 64)],
        table.at[pl.ds(pl.multiple_of(NG + NC + w * 64, 8), 64)])


@functools.partial(
    pl.kernel,
    mesh=_mesh(),
    compiler_params=_cparams(),
    out_type=jax.ShapeDtypeStruct((96 * 96 * 2048,), jnp.float32),
    scratch_types=[
        pltpu.VMEM((2 * NW * 32,), jnp.int32),
        pltpu.VMEM((WTBL,), jnp.int32),
        pltpu.VMEM((SEGCHUNK,), jnp.int32),
        pltpu.VMEM((CHN,), jnp.int32),
        pltpu.VMEM((CHN,), jnp.int32),
        pltpu.VMEM((CHN, CH), jnp.float32),
        pltpu.VMEM((CHN, CH), jnp.float32),
        pltpu.VMEM((PCH * 2048,), jnp.float32),
        pltpu.SemaphoreType.DMA,
        pltpu.SemaphoreType.DMA,
    ],
)
def _kernel_b(part_g, part_c, counts, table, out,
              counts_v, winner_v, seg_v, idx0_v, idx1_v, rows0_v, rows1_v,
              pen_v, sem0, sem1):
    w = lax.axis_index("s") * 2 + lax.axis_index("c")
    lane = _iota()
    pltpu.sync_copy(counts, counts_v)

    def wzero(i, _):
        winner_v[pl.ds(i * 16, 16)] = _splat(-1)
        return 0
    lax.fori_loop(0, WTBL // 16, wzero, 0)

    w16 = _splat(w)

    def ingest_kind(kind, ROW, vr_base_mul, vr_base_add, part_hbm):
        def per_src(src, _):
            b = kind * (NW * 32) + src * 32
            r0 = counts_v[pl.ds(b, 16)]
            r1 = counts_v[pl.ds(b + 16, 16)]
            rnd0 = jnp.bitwise_and(r0 + 15, _splat(~15))
            rnd1 = jnp.bitwise_and(r1 + 15, _splat(~15))
            n = (jnp.sum(jnp.where(lane == w16, r0, 0))
                 + jnp.sum(jnp.where(lane + 16 == w16, r1, 0)))
            off = (jnp.sum(jnp.where(lane < w16, rnd0, 0))
                   + jnp.sum(jnp.where(lane + 16 < w16, rnd1, 0)))
            n16 = jnp.bitwise_and(n + 15, jnp.int32(~15))
            vr_base = src * vr_base_mul + vr_base_add
            nchunks = (n16 + (SEGCHUNK - 1)) // SEGCHUNK

            def per_chunk(c, _):
                pltpu.sync_copy(
                    part_hbm.at[pl.ds(
                        pl.multiple_of(src * ROW + off + c * SEGCHUNK, 8),
                        SEGCHUNK)],
                    seg_v)
                svr = jnp.minimum(SEGCHUNK, n16 - c * SEGCHUNK) // 16

                def per_vreg(j, _):
                    e = seg_v[pl.ds(j * 16, 16)]
                    rl = jnp.bitwise_and(e, _splat(32767))
                    lsq = lax.shift_right_logical(e, _splat(15))
                    key = jnp.bitwise_or(lax.shift_left(rl, _splat(14)), lsq)
                    vr = _splat(vr_base) + lsq
                    sk, sv = plsc.sort_key_val(key, vr)
                    rls = lax.shift_right_logical(sk, _splat(14))
                    old = plsc.load_gather(winner_v, [rls])
                    plsc.store_scatter(winner_v, [rls], jnp.maximum(old, sv))
                    return 0
                lax.fori_loop(0, svr, per_vreg, 0)
                return 0
            lax.fori_loop(0, nchunks, per_chunk, 0)
            return 0
        lax.fori_loop(0, NW, per_src, 0)

    ingest_kind(0, GROW, GB, 0, part_g)
    ingest_kind(1, CROW, CB, NG, part_c)

    def build_idx_chunk(c, idx_ref):
        def build_idx(v, _):
            wv = winner_v[pl.ds(c * CHN + v * 16, 16)]
            pos = _splat(c * CHN + v * 16) + lane
            dummy = _splat(NG + NC) + jnp.bitwise_and(pos + w16 * 64,
                                                      _splat(NDUMMY - 1))
            idx_ref[pl.ds(v * 16, 16)] = jnp.where(wv < 0, dummy, wv)
            return 0
        lax.fori_loop(0, CHN // 16, build_idx, 0)

    idx_bufs = (idx0_v, idx1_v)
    row_bufs = (rows0_v, rows1_v)
    sems = (sem0, sem1)
    lane16 = lane

    def fire(c):
        b = c % 2
        build_idx_chunk(c, idx_bufs[b])
        return pltpu.async_copy(table.at[idx_bufs[b]], row_bufs[b], sems[b])

    handle = fire(0)
    for c in range(18):
        b = c % 2
        handle.wait()
        if c + 1 < 18:
            handle = fire(c + 1)
        rows_ref = row_bufs[b]

        def xpose_p(p, _):
            def xpose_c(ch, _):
                rbase = p * 96 + lane16
                for zg in range(6):
                    vals = plsc.load_gather(
                        rows_ref, [rbase + _splat(zg * 16), _splat(ch)])
                    pen_v[pl.ds(p * 2048 + ch * 128 + zg * 16, 16)] = vals
                return 0
            lax.fori_loop(0, CH, xpose_c, 0)
            return 0
        lax.fori_loop(0, PCH, xpose_p, 0)

        off = pl.multiple_of((w * 288 + c * PCH) * 2048, 8)
        pltpu.sync_copy(pen_v, out.at[pl.ds(off, PCH * 2048)])


def kernel(current_values, global_values, current_coords, global_coords,
           relative_origin):
    origin_b = jnp.broadcast_to(
        relative_origin.astype(jnp.int32)[:, None], (3, 16)).reshape(48)
    part_g, part_c, counts, table = _kernel_a(
        current_coords.astype(jnp.int32).T.reshape(-1),
        global_coords.astype(jnp.int32).T.reshape(-1), origin_b,
        current_values, global_values)
    outf = _kernel_b(part_g, part_c, counts, table)
    return outf.reshape(96, 96, 2, 8, 128)[:, :, :, :, :96].transpose(
        0, 1, 4, 2, 3).reshape(96, 96, 96, CH)


# speedup vs baseline: 10.6356x; 1.0966x over previous
import functools

import jax
import jax.numpy as jnp
from jax import lax
from jax.experimental import pallas as pl
from jax.experimental.pallas import tpu as pltpu
from jax.experimental.pallas import tpu_sc as plsc

V = 96 * 96 * 96
NG = 524288
NC = 262144
CH = 16

NW = 32
GB = NG // NW
CB = NC // NW
SLAB = V // NW
WTBL = 32768

GSTAGE = GB + 16 * NW
CSTAGE = CB + 16 * NW
SEGCHUNK = 2048
GROW = GSTAGE + SEGCHUNK
CROW = CSTAGE + SEGCHUNK
PCH = 16
CHN = PCH * 96
NDUMMY = 2048
SENT = 32767

_mesh = lambda: plsc.VectorSubcoreMesh(core_axis_name="c", subcore_axis_name="s")
_cparams = lambda: pltpu.CompilerParams(needs_layout_passes=False,
                                        use_tc_tiling_on_sc=False)


def _iota():
    return lax.iota(jnp.int32, 16)


def _splat(x):
    return jnp.broadcast_to(jnp.asarray(x, jnp.int32), (16,))


@functools.partial(
    pl.kernel,
    mesh=_mesh(),
    compiler_params=_cparams(),
    out_type=jax.ShapeDtypeStruct((NG + NC + NDUMMY, CH), jnp.float32),
    scratch_types=[pltpu.VMEM((2048, CH), jnp.float32)],
)
def _kernel_a1(cur_vals, glob_vals, table, vbuf_v):
    w = lax.axis_index("s") * 2 + lax.axis_index("c")

    def copy_vals(vals_hbm, src_base, dst_base, nchunks):
        def cp(i, _):
            so = pl.multiple_of(src_base + i * 2048, 8)
            do = pl.multiple_of(dst_base + i * 2048, 8)
            pltpu.sync_copy(vals_hbm.at[pl.ds(so, 2048)], vbuf_v)
            pltpu.sync_copy(vbuf_v, table.at[pl.ds(do, 2048)])
            return 0
        lax.fori_loop(0, nchunks, cp, 0)

    copy_vals(glob_vals, w * GB, w * GB, GB // 2048)
    copy_vals(cur_vals, w * CB, NG + w * CB, CB // 2048)

    zrow = jnp.zeros((16,), jnp.float32)

    def zfill(i, _):
        vbuf_v[i, :] = zrow
        return 0
    lax.fori_loop(0, 64, zfill, 0)
    pltpu.sync_copy(
        vbuf_v.at[pl.ds(0, 64)],
        table.at[pl.ds(pl.multiple_of(NG + NC + w * 64, 8), 64)])


@functools.partial(
    pl.kernel,
    mesh=_mesh(),
    compiler_params=_cparams(),
    out_type=(
        jax.ShapeDtypeStruct((NW * GROW,), jnp.int32),
        jax.ShapeDtypeStruct((NW * CROW,), jnp.int32),
        jax.ShapeDtypeStruct((2 * NW * 32,), jnp.int32),
    ),
    scratch_types=[
        pltpu.VMEM((GB * 3,), jnp.int32),
        pltpu.VMEM((GB,), jnp.int32),
        pltpu.VMEM((GSTAGE,), jnp.int32),
        pltpu.VMEM((512,), jnp.int32),
        pltpu.VMEM((512,), jnp.int32),
        pltpu.VMEM((32,), jnp.int32),
        pltpu.VMEM((48,), jnp.int32),
    ],
)
def _kernel_a2(cur_coords, glob_coords, origin_b,
               part_g, part_c, counts,
               coords_v, rbuf_v, stage_v, hist_v, colptr_v, counts_v,
               origin_v):
    w = lax.axis_index("s") * 2 + lax.axis_index("c")
    lane = _iota()
    zeros16 = _splat(0)
    pltpu.sync_copy(origin_b, origin_v)

    def run_kind(kind, B, BSTAGE, ROW, coords_hbm, part_hbm, shift_origin, N):
        nvr = B // 16
        for c in range(3):
            pltpu.sync_copy(
                coords_hbm.at[pl.ds(pl.multiple_of(c * N + w * B, 8), B)],
                coords_v.at[pl.ds(c * B, B)])

        if shift_origin:
            ox = origin_v[pl.ds(0, 16)]
            oy = origin_v[pl.ds(16, 16)]
            oz = origin_v[pl.ds(32, 16)]

        def zero_hist(i, _):
            hist_v[pl.ds(i * 16, 16)] = zeros16
            return 0
        lax.fori_loop(0, 32, zero_hist, 0)

        def pass_a(i, _):
            x = coords_v[pl.ds(i * 16, 16)]
            y = coords_v[pl.ds(B + i * 16, 16)]
            z = coords_v[pl.ds(2 * B + i * 16, 16)]
            if shift_origin:
                x = x - ox
                y = y - oy
                z = z - oz
                valid = ((x >= 0) & (x < 96) & (y >= 0) & (y < 96)
                         & (z >= 0) & (z < 96))
                r = (x * 96 + y) * 96 + z
                renc = jnp.where(valid, r, _splat(-1))
            else:
                renc = (x * 96 + y) * 96 + z
                valid = None
            rbuf_v[pl.ds(i * 16, 16)] = renc
            owner = jnp.where(renc >= 0, renc, 0) // SLAB
            col = owner * 16 + lane
            if valid is None:
                plsc.addupdate_scatter(hist_v, [col], _splat(1))
            else:
                plsc.addupdate_scatter(hist_v, [col], _splat(1), mask=valid)
            return 0
        lax.fori_loop(0, nvr, pass_a, 0)

        lane0 = lane == 0

        def prefix(o, base):
            h = hist_v[pl.ds(o * 16, 16)]
            incl = plsc.cumsum(h)
            tot = jnp.sum(h)
            colptr_v[pl.ds(o * 16, 16)] = _splat(base) + (incl - h)
            plsc.store_scatter(counts_v, [_splat(o)], _splat(tot), mask=lane0)
            nbase = base + tot
            return jnp.bitwise_and(nbase + 15, jnp.int32(~15))
        lax.fori_loop(0, 32, prefix, jnp.int32(0))

        def fill(i, _):
            stage_v[pl.ds(i * 16, 16)] = _splat(SENT)
            return 0
        lax.fori_loop(0, BSTAGE // 16, fill, 0)

        def pass_b(i, _):
            renc = rbuf_v[pl.ds(i * 16, 16)]
            valid = renc >= 0
            rr = jnp.where(valid, renc, 0)
            owner = rr // SLAB
            rl = rr - owner * SLAB
            lseq = _splat(i * 16) + lane
            payload = jnp.bitwise_or(lax.shift_left(lseq, _splat(15)), rl)
            col = owner * 16 + lane
            pos = plsc.load_gather(colptr_v, [col])
            plsc.store_scatter(stage_v, [pos], payload, mask=valid)
            plsc.store_scatter(colptr_v, [col], pos + 1, mask=valid)
            return 0
        lax.fori_loop(0, nvr, pass_b, 0)

        pltpu.sync_copy(stage_v.at[pl.ds(0, BSTAGE)],
                        part_hbm.at[pl.ds(pl.multiple_of(w * ROW, 8), BSTAGE)])
        pltpu.sync_copy(
            counts_v,
            counts.at[pl.ds(pl.multiple_of(kind * (NW * 32) + w * 32, 8), 32)])

    run_kind(0, GB, GSTAGE, GROW, glob_coords, part_g, True, NG)
    run_kind(1, CB, CSTAGE, CROW, cur_coords, part_c, False, NC)


@functools.partial(
    pl.kernel,
    mesh=_mesh(),
    compiler_params=_cparams(),
    out_type=jax.ShapeDtypeStruct((96 * 96 * 2048,), jnp.float32),
    scratch_types=[
        pltpu.VMEM((2 * NW * 32,), jnp.int32),
        pltpu.VMEM((WTBL,), jnp.int32),
        pltpu.VMEM((SEGCHUNK,), jnp.int32),
        pltpu.VMEM((CHN,), jnp.int32),
        pltpu.VMEM((CHN,), jnp.int32),
        pltpu.VMEM((CHN, CH), jnp.float32),
        pltpu.VMEM((CHN, CH), jnp.float32),
        pltpu.VMEM((PCH * 2048,), jnp.float32),
        pltpu.SemaphoreType.DMA,
        pltpu.SemaphoreType.DMA,
    ],
)
def _kernel_b(part_g, part_c, counts, table, out,
              counts_v, winner_v, seg_v, idx0_v, idx1_v, rows0_v, rows1_v,
              pen_v, sem0, sem1):
    w = lax.axis_index("s") * 2 + lax.axis_index("c")
    lane = _iota()
    pltpu.sync_copy(counts, counts_v)

    def wzero(i, _):
        winner_v[pl.ds(i * 16, 16)] = _splat(-1)
        return 0
    lax.fori_loop(0, WTBL // 16, wzero, 0)

    w16 = _splat(w)

    def ingest_kind(kind, ROW, vr_base_mul, vr_base_add, part_hbm):
        def per_src(src, _):
            b = kind * (NW * 32) + src * 32
            r0 = counts_v[pl.ds(b, 16)]
            r1 = counts_v[pl.ds(b + 16, 16)]
            rnd0 = jnp.bitwise_and(r0 + 15, _splat(~15))
            rnd1 = jnp.bitwise_and(r1 + 15, _splat(~15))
            n = (jnp.sum(jnp.where(lane == w16, r0, 0))
                 + jnp.sum(jnp.where(lane + 16 == w16, r1, 0)))
            off = (jnp.sum(jnp.where(lane < w16, rnd0, 0))
                   + jnp.sum(jnp.where(lane + 16 < w16, rnd1, 0)))
            n16 = jnp.bitwise_and(n + 15, jnp.int32(~15))
            vr_base = src * vr_base_mul + vr_base_add
            nchunks = (n16 + (SEGCHUNK - 1)) // SEGCHUNK

            def per_chunk(c, _):
                pltpu.sync_copy(
                    part_hbm.at[pl.ds(
                        pl.multiple_of(src * ROW + off + c * SEGCHUNK, 8),
                        SEGCHUNK)],
                    seg_v)
                svr = jnp.minimum(SEGCHUNK, n16 - c * SEGCHUNK) // 16

                def per_vreg(j, _):
                    e = seg_v[pl.ds(j * 16, 16)]
                    rl = jnp.bitwise_and(e, _splat(32767))
                    lsq = lax.shift_right_logical(e, _splat(15))
                    key = jnp.bitwise_or(lax.shift_left(rl, _splat(14)), lsq)
                    vr = _splat(vr_base) + lsq
                    sk, sv = plsc.sort_key_val(key, vr)
                    rls = lax.shift_right_logical(sk, _splat(14))
                    old = plsc.load_gather(winner_v, [rls])
                    plsc.store_scatter(winner_v, [rls], jnp.maximum(old, sv))
                    return 0
                lax.fori_loop(0, svr, per_vreg, 0)
                return 0
            lax.fori_loop(0, nchunks, per_chunk, 0)
            return 0
        lax.fori_loop(0, NW, per_src, 0)

    ingest_kind(0, GROW, GB, 0, part_g)
    ingest_kind(1, CROW, CB, NG, part_c)

    def build_idx_chunk(c, idx_ref):
        def build_idx(v, _):
            wv = winner_v[pl.ds(c * CHN + v * 16, 16)]
            pos = _splat(c * CHN + v * 16) + lane
            dummy = _splat(NG + NC) + jnp.bitwise_and(pos + w16 * 64,
                                                      _splat(NDUMMY - 1))
            idx_ref[pl.ds(v * 16, 16)] = jnp.where(wv < 0, dummy, wv)
            return 0
        lax.fori_loop(0, CHN // 16, build_idx, 0)

    idx_bufs = (idx0_v, idx1_v)
    row_bufs = (rows0_v, rows1_v)
    sems = (sem0, sem1)
    lane16 = lane

    def fire(c):
        b = c % 2
        build_idx_chunk(c, idx_bufs[b])
        return pltpu.async_copy(table.at[idx_bufs[b]], row_bufs[b], sems[b])

    handle = fire(0)
    for c in range(18):
        b = c % 2
        handle.wait()
        if c + 1 < 18:
            handle = fire(c + 1)
        rows_ref = row_bufs[b]

        def xpose_p(p, _):
            def xpose_c(ch, _):
                rbase = p * 96 + lane16
                for zg in range(6):
                    vals = plsc.load_gather(
                        rows_ref, [rbase + _splat(zg * 16), _splat(ch)])
                    pen_v[pl.ds(p * 2048 + ch * 128 + zg * 16, 16)] = vals
                return 0
            lax.fori_loop(0, CH, xpose_c, 0)
            return 0
        lax.fori_loop(0, PCH, xpose_p, 0)

        off = pl.multiple_of((w * 288 + c * PCH) * 2048, 8)
        pltpu.sync_copy(pen_v, out.at[pl.ds(off, PCH * 2048)])


def kernel(current_values, global_values, current_coords, global_coords,
           relative_origin):
    origin_b = jnp.broadcast_to(
        relative_origin.astype(jnp.int32)[:, None], (3, 16)).reshape(48)
    table = _kernel_a1(current_values, global_values)
    part_g, part_c, counts = _kernel_a2(
        current_coords.astype(jnp.int32).T.reshape(-1),
        global_coords.astype(jnp.int32).T.reshape(-1), origin_b)
    outf = _kernel_b(part_g, part_c, counts, table)
    return outf.reshape(96, 96, 2, 8, 128)[:, :, :, :, :96].transpose(
        0, 1, 4, 2, 3).reshape(96, 96, 96, CH)
